# Initial kernel scaffold; baseline (speedup 1.0000x reference)
import jax
import jax.numpy as jnp
from jax.experimental import pallas as pl


def kernel(token_ids, W):
    raise NotImplementedError("write your pallas kernel here")


# speedup vs baseline: 1.8164x; 1.8164x over previous
import functools

import jax
import jax.numpy as jnp
from jax import lax
from jax.experimental import pallas as pl
from jax.experimental.pallas import tpu as pltpu
from jax.experimental.pallas import tpu_sc as plsc

NUM_EMB = 1_000_000
DIM = 64
BATCH = 16384
HIST = 50

NC = 2
NS = 16
NW = NC * NS

TOTAL = BATCH * HIST
PER_W = TOTAL // NW
GATHER = 128
K = 5
CHUNK = K * GATHER
CHUNKS_PER_W = PER_W // CHUNK
G = NW * CHUNKS_PER_W


def _mesh():
    return plsc.VectorSubcoreMesh(core_axis_name="c", subcore_axis_name="s")


@functools.partial(
    pl.kernel,
    out_type=jax.ShapeDtypeStruct((G, CHUNK, DIM), jnp.float32),
    mesh=_mesh(),
    compiler_params=pltpu.CompilerParams(use_tc_tiling_on_sc=False),
    scratch_types=[
        pltpu.VMEM((K, GATHER), jnp.int32),
        pltpu.VMEM((CHUNK, DIM), jnp.float32),
        pltpu.SemaphoreType.DMA,
    ],
)
def _gather_kernel(idx_hbm, table_hbm, out_hbm, idx_v, rows_v, sem):
    wid = lax.axis_index("s") * NC + lax.axis_index("c")
    g0 = wid * CHUNKS_PER_W

    @pl.loop(0, CHUNKS_PER_W)
    def _(c):
        g = g0 + c
        pltpu.sync_copy(idx_hbm.at[g], idx_v)
        copies = [
            pltpu.async_copy(
                table_hbm.at[idx_v.at[j]],
                rows_v.at[pl.ds(j * GATHER, GATHER)],
                sem,
            )
            for j in range(K)
        ]
        for cp in copies:
            cp.wait()
        pltpu.sync_copy(rows_v, out_hbm.at[g])


def kernel(token_ids, W):
    idx = token_ids.astype(jnp.int32).reshape(G, K, GATHER)
    out = _gather_kernel(idx, W)
    return out.reshape(BATCH, HIST, DIM)


# speedup vs baseline: 1.8739x; 1.0317x over previous
import functools

import jax
import jax.numpy as jnp
from jax import lax
from jax.experimental import pallas as pl
from jax.experimental.pallas import tpu as pltpu
from jax.experimental.pallas import tpu_sc as plsc

NUM_EMB = 1_000_000
DIM = 64
BATCH = 16384
HIST = 50

NC = 2
NS = 16
NW = NC * NS

TOTAL = BATCH * HIST
PER_W = TOTAL // NW
GATHER = 128
K = 5
CHUNK = K * GATHER
CHUNKS_PER_W = PER_W // CHUNK
G = NW * CHUNKS_PER_W


def _mesh():
    return plsc.VectorSubcoreMesh(core_axis_name="c", subcore_axis_name="s")


@functools.partial(
    pl.kernel,
    out_type=jax.ShapeDtypeStruct((G, CHUNK, DIM), jnp.float32),
    mesh=_mesh(),
    compiler_params=pltpu.CompilerParams(use_tc_tiling_on_sc=False),
    scratch_types=[
        pltpu.VMEM((CHUNKS_PER_W * K, GATHER), jnp.int32),
        pltpu.VMEM((CHUNK, DIM), jnp.float32),
        pltpu.VMEM((CHUNK, DIM), jnp.float32),
        pltpu.SemaphoreType.DMA,
        pltpu.SemaphoreType.DMA,
        pltpu.SemaphoreType.DMA,
        pltpu.SemaphoreType.DMA,
    ],
)
def _gather_kernel(idx_hbm, table_hbm, out_hbm, idx_all, rows0, rows1,
                   g0sem, g1sem, o0sem, o1sem):
    wid = lax.axis_index("s") * NC + lax.axis_index("c")
    base = wid * CHUNKS_PER_W
    pltpu.sync_copy(idx_hbm.at[wid], idx_all)

    rows = (rows0, rows1)
    gsem = (g0sem, g1sem)
    osem = (o0sem, o1sem)

    def fire(chunk, slot):
        for j in range(K):
            pltpu.async_copy(
                table_hbm.at[idx_all.at[chunk * K + j]],
                rows[slot].at[pl.ds(j * GATHER, GATHER)],
                gsem[slot],
            )

    def drain_gathers(slot):
        pltpu.make_async_copy(out_hbm.at[0], rows[slot], gsem[slot]).wait()

    def out_start(chunk, slot):
        pltpu.async_copy(rows[slot], out_hbm.at[base + chunk], osem[slot])

    def drain_out(slot):
        pltpu.make_async_copy(out_hbm.at[0], rows[slot], osem[slot]).wait()

    fire(0, 0)

    @pl.loop(0, CHUNKS_PER_W, step=2)
    def _(c):
        @pl.when(c > 0)
        def _():
            drain_out(1)

        fire(c + 1, 1)
        drain_gathers(0)
        out_start(c, 0)

        @pl.when(c + 2 < CHUNKS_PER_W)
        def _():
            drain_out(0)
            fire(c + 2, 0)

        drain_gathers(1)
        out_start(c + 1, 1)

    drain_out(0)
    drain_out(1)


def kernel(token_ids, W):
    idx = token_ids.astype(jnp.int32).reshape(NW, CHUNKS_PER_W * K, GATHER)
    out = _gather_kernel(idx, W)
    return out.reshape(BATCH, HIST, DIM)
